# baseline (speedup 1.0000x reference)
import functools

import jax
import jax.numpy as jnp
from jax import lax
from jax.experimental import pallas as pl
from jax.experimental.pallas import tpu as pltpu
from jax.experimental.pallas import tpu_sc as plsc

N = 10000
E = 320000
FDIM = 128
CDIM = 16
HDIM = 128
LDIM = 64

NC = 2
NS = 16
NW = NC * NS
E_PER = E // NW
K = 40
ITERS = E_PER // K
NB = 8
MAIN = (ITERS // NB) * NB
ROWB = 624
ROWB_LAST = N - (NS - 1) * ROWB
ZRB = 48

_f32 = jnp.float32


def _mesh():
    return plsc.VectorSubcoreMesh(core_axis_name="c", subcore_axis_name="s")


def _fill_zeros(zbuf, n_rows, C):
    def fz(i, _):
        def fz2(j, _):
            zbuf[i, pl.ds(j * 16, 16)] = jnp.zeros((16,), _f32)
            return 0

        lax.fori_loop(0, C // 16, fz2, 0)
        return 0

    lax.fori_loop(0, n_rows, fz, 0)


def _zero_acc(zbuf, acc, s):
    base_r = s * ROWB
    for i in range(ROWB // ZRB):
        pltpu.sync_copy(zbuf, acc.at[pl.ds(base_r + i * ZRB, ZRB)])

    @pl.when(s == NS - 1)
    def _():
        pltpu.sync_copy(zbuf.at[pl.ds(0, ROWB_LAST - ROWB)],
                        acc.at[pl.ds(base_r + ROWB, ROWB_LAST - ROWB)])


def _flush_acc(acc, out_hbm, c, s):
    base_r = s * ROWB

    @pl.when(s == NS - 1)
    def _():
        pltpu.sync_copy(acc.at[pl.ds(base_r, ROWB_LAST)],
                        out_hbm.at[c, pl.ds(base_r, ROWB_LAST)])

    @pl.when(s != NS - 1)
    def _():
        pltpu.sync_copy(acc.at[pl.ds(base_r, ROWB)],
                        out_hbm.at[c, pl.ds(base_r, ROWB)])


def _sc_degree(dst_hbm_arg):
    C = 128

    @functools.partial(
        pl.kernel,
        out_type=jax.ShapeDtypeStruct((NC, N, C), _f32),
        mesh=_mesh(),
        scratch_types=(
            [pltpu.VMEM((K,), jnp.int32) for _ in range(NB)]
            + [pltpu.VMEM((K, C), _f32), pltpu.VMEM((K, C), _f32)]
            + [pltpu.SemaphoreType.DMA for _ in range(2 * NB)]
            + [pltpu.VMEM_SHARED((N, C), _f32)]
        ),
    )
    def deg_kernel(dst_hbm, out_hbm, *scr):
        dv = scr[0:NB]
        ones_v = scr[NB]
        zbuf = scr[NB + 1]
        isem = scr[NB + 2:NB + 2 + NB]
        ssem = scr[NB + 2 + NB:NB + 2 + 2 * NB]
        acc = scr[-1]
        c = lax.axis_index("c")
        s = lax.axis_index("s")
        wid = c * NS + s

        def fill(i, _):
            def fill2(j, _):
                ones_v[i, pl.ds(j * 16, 16)] = jnp.ones((16,), _f32)
                return 0

            lax.fori_loop(0, C // 16, fill2, 0)
            return 0

        lax.fori_loop(0, K, fill, 0)
        _fill_zeros(zbuf, K, C)
        base_r = s * ROWB
        full = ROWB // K
        zd = []
        for i in range(full):
            zd.append(pltpu.async_copy(
                zbuf, acc.at[pl.ds(base_r + i * K, K)], isem[0]))
        for d in zd:
            d.wait()

        @pl.when(s == NS - 1)
        def _():
            for j in range((ROWB_LAST - full * K) // K):
                pltpu.sync_copy(zbuf,
                                acc.at[pl.ds(base_r + (full + j) * K, K)])

        @pl.when(s != NS - 1)
        def _():
            rem = ROWB - full * K
            if rem:
                pltpu.sync_copy(zbuf.at[pl.ds(0, rem)],
                                acc.at[pl.ds(base_r + full * K, rem)])

        plsc.subcore_barrier()

        base_e = wid * E_PER

        def body(g, _):
            ic = []
            for i in range(NB):
                ch = NB * g + i
                ic.append(pltpu.async_copy(
                    dst_hbm.at[pl.ds(base_e + ch * K, K)], dv[i], isem[i]))
            sc = []
            for i in range(NB):
                ic[i].wait()
                sc.append(pltpu.async_copy(ones_v, acc.at[dv[i]], ssem[i],
                                           add=True))
            for i in range(NB):
                sc[i].wait()
            return 0

        lax.fori_loop(0, ITERS // NB, body, 0)
        for ch in range(MAIN, ITERS):
            pltpu.sync_copy(dst_hbm.at[pl.ds(base_e + ch * K, K)], dv[0])
            pltpu.sync_copy(ones_v, acc.at[dv[0]], add=True)
        plsc.subcore_barrier()
        _flush_acc(acc, out_hbm, c, s)

    return deg_kernel(dst_hbm_arg)


def _make_prop(C):
    @functools.partial(
        pl.kernel,
        out_type=jax.ShapeDtypeStruct((NC, N, C), _f32),
        mesh=_mesh(),
        scratch_types=(
            [pltpu.VMEM((K,), jnp.int32) for _ in range(2 * NB)]
            + [pltpu.VMEM((K, C), _f32) for _ in range(NB)]
            + [pltpu.SemaphoreType.DMA for _ in range(3 * NB)]
            + [pltpu.VMEM_SHARED((N, C), _f32)]
        ),
    )
    def prop(y_hbm, src_hbm, dst_hbm, out_hbm, *scr):
        sv = scr[0:NB]
        dv = scr[NB:2 * NB]
        rows = scr[2 * NB:3 * NB]
        isem = scr[3 * NB:3 * NB + NB]
        gsem = scr[3 * NB + NB:3 * NB + 2 * NB]
        ssem = scr[3 * NB + 2 * NB:3 * NB + 3 * NB]
        acc = scr[-1]
        c = lax.axis_index("c")
        s = lax.axis_index("s")
        wid = c * NS + s

        _fill_zeros(rows[0], K, C)
        base_r = s * ROWB
        full = ROWB // K
        zd = []
        for i in range(full):
            zd.append(pltpu.async_copy(
                rows[0], acc.at[pl.ds(base_r + i * K, K)], gsem[0]))
        for d in zd:
            d.wait()

        @pl.when(s == NS - 1)
        def _():
            for j in range((ROWB_LAST - full * K) // K):
                pltpu.sync_copy(rows[0],
                                acc.at[pl.ds(base_r + (full + j) * K, K)])

        @pl.when(s != NS - 1)
        def _():
            rem = ROWB - full * K
            if rem:
                pltpu.sync_copy(rows[0].at[pl.ds(0, rem)],
                                acc.at[pl.ds(base_r + full * K, rem)])

        plsc.subcore_barrier()

        base_e = wid * E_PER

        def body(g, _):
            ic = []
            for i in range(NB):
                ch = NB * g + i
                off = base_e + ch * K
                i1 = pltpu.async_copy(src_hbm.at[pl.ds(off, K)], sv[i],
                                      isem[i])
                i2 = pltpu.async_copy(dst_hbm.at[pl.ds(off, K)], dv[i],
                                      isem[i])
                ic.append((i1, i2))
            gc = []
            for i in range(NB):
                ic[i][0].wait()
                ic[i][1].wait()
                gc.append(pltpu.async_copy(y_hbm.at[sv[i]], rows[i], gsem[i]))
            sc = []
            for i in range(NB):
                gc[i].wait()
                sc.append(pltpu.async_copy(rows[i], acc.at[dv[i]], ssem[i],
                                           add=True))
            for i in range(NB):
                sc[i].wait()
            return 0

        lax.fori_loop(0, ITERS // NB, body, 0)
        for ch in range(MAIN, ITERS):
            off = base_e + ch * K
            pltpu.sync_copy(src_hbm.at[pl.ds(off, K)], sv[0])
            pltpu.sync_copy(dst_hbm.at[pl.ds(off, K)], dv[0])
            pltpu.async_copy(y_hbm.at[sv[0]], rows[0], gsem[0]).wait()
            pltpu.sync_copy(rows[0], acc.at[dv[0]], add=True)
        plsc.subcore_barrier()
        _flush_acc(acc, out_hbm, c, s)

    return prop


def _make_prop_dual(C):
    E_PER2 = E // NS
    ITERS2 = E_PER2 // K
    MAIN2 = (ITERS2 // NB) * NB

    @functools.partial(
        pl.kernel,
        out_type=jax.ShapeDtypeStruct((NC, N, C), _f32),
        mesh=_mesh(),
        scratch_types=(
            [pltpu.VMEM((K,), jnp.int32) for _ in range(2 * NB)]
            + [pltpu.VMEM((K, C), _f32) for _ in range(NB)]
            + [pltpu.SemaphoreType.DMA for _ in range(3 * NB)]
            + [pltpu.VMEM_SHARED((N, C), _f32)]
        ),
    )
    def prop2(y_hbm, src_hbm, dst_hbm, out_hbm, *scr):
        sv = scr[0:NB]
        dv = scr[NB:2 * NB]
        rows = scr[2 * NB:3 * NB]
        isem = scr[3 * NB:3 * NB + NB]
        gsem = scr[3 * NB + NB:3 * NB + 2 * NB]
        ssem = scr[3 * NB + 2 * NB:3 * NB + 3 * NB]
        acc = scr[-1]
        c = lax.axis_index("c")
        s = lax.axis_index("s")

        _fill_zeros(rows[0], K, C)
        base_r = s * ROWB
        full = ROWB // K
        zd = []
        for i in range(full):
            zd.append(pltpu.async_copy(
                rows[0], acc.at[pl.ds(base_r + i * K, K)], gsem[0]))
        for d in zd:
            d.wait()

        @pl.when(s == NS - 1)
        def _():
            for j in range((ROWB_LAST - full * K) // K):
                pltpu.sync_copy(rows[0],
                                acc.at[pl.ds(base_r + (full + j) * K, K)])

        @pl.when(s != NS - 1)
        def _():
            rem = ROWB - full * K
            if rem:
                pltpu.sync_copy(rows[0].at[pl.ds(0, rem)],
                                acc.at[pl.ds(base_r + full * K, rem)])

        plsc.subcore_barrier()

        base_e = s * E_PER2
        ypanel = y_hbm.at[c]

        def body(g, _):
            ic = []
            for i in range(NB):
                ch = NB * g + i
                off = base_e + ch * K
                i1 = pltpu.async_copy(src_hbm.at[pl.ds(off, K)], sv[i],
                                      isem[i])
                i2 = pltpu.async_copy(dst_hbm.at[pl.ds(off, K)], dv[i],
                                      isem[i])
                ic.append((i1, i2))
            gc = []
            for i in range(NB):
                ic[i][0].wait()
                ic[i][1].wait()
                gc.append(pltpu.async_copy(ypanel.at[sv[i]], rows[i],
                                           gsem[i]))
            sc = []
            for i in range(NB):
                gc[i].wait()
                sc.append(pltpu.async_copy(rows[i], acc.at[dv[i]], ssem[i],
                                           add=True))
            for i in range(NB):
                sc[i].wait()
            return 0

        lax.fori_loop(0, ITERS2 // NB, body, 0)
        for ch in range(MAIN2, ITERS2):
            off = base_e + ch * K
            pltpu.sync_copy(src_hbm.at[pl.ds(off, K)], sv[0])
            pltpu.sync_copy(dst_hbm.at[pl.ds(off, K)], dv[0])
            pltpu.async_copy(ypanel.at[sv[0]], rows[0], gsem[0]).wait()
            pltpu.sync_copy(rows[0], acc.at[dv[0]], add=True)
        plsc.subcore_barrier()
        _flush_acc(acc, out_hbm, c, s)

    return prop2


_prop128 = _make_prop(HDIM)
_prop_dual128 = _make_prop_dual(HDIM)


def _tc_a_body(degp, f, cnd, w1, w2, dis_o, y12_o):
    dp = degp[...]
    deg = dp[0, :, 0:1] + dp[1, :, 0:1] + 1.0
    dis = lax.rsqrt(deg)
    dis_o[...] = dis
    y12_o[0] = dis * jnp.dot(f[...], w1[...], preferred_element_type=_f32)
    y12_o[1] = dis * jnp.dot(cnd[...], w2[...], preferred_element_type=_f32)


def _tc_b_body(agg, y12, dis, bf, bc, w, y3_o):
    d = dis[...]
    a = agg[...]
    y = y12[...]
    ha = jnp.tanh(d * (a[0] + y[0]) + bf[...])
    hb = jnp.tanh(d * (a[1] + y[1]) + bc[...])
    wm = w[...]
    y3_o[...] = d * (jnp.dot(ha, wm[:HDIM], preferred_element_type=_f32)
                     + jnp.dot(hb, wm[HDIM:], preferred_element_type=_f32))


def _tc_c_body(a0, a1, y3, dis, bh, w, y4_o):
    d = dis[...]
    h3 = jnp.tanh(d * (a0[...] + a1[...] + y3[...]) + bh[...])
    y4 = d * jnp.dot(h3, w[...], preferred_element_type=_f32)
    y4_o[...] = jnp.concatenate([y4, jnp.zeros((N, HDIM - LDIM), _f32)], axis=1)


def _tc_d_body(a0, a1, y4, dis, bl, z_o):
    d = dis[...]
    z_o[...] = (d * (a0[...] + a1[...] + y4[...]))[:, :LDIM] + bl[...]


def kernel(feature, condition, edge_index, W_f2h, b_f2h, W_c2h, b_c2h,
           W_h2h, b_h2h, W_h2l, b_h2l):
    src1 = edge_index[0].astype(jnp.int32)
    dst1 = edge_index[1].astype(jnp.int32)
    bf = b_f2h.reshape(1, HDIM)
    bc = b_c2h.reshape(1, HDIM)
    bh = b_h2h.reshape(1, HDIM)
    bl = b_h2l.reshape(1, LDIM)

    degp = _sc_degree(dst1)

    dis, y12 = pl.pallas_call(
        _tc_a_body,
        out_shape=(
            jax.ShapeDtypeStruct((N, 1), _f32),
            jax.ShapeDtypeStruct((NC, N, HDIM), _f32),
        ),
    )(degp, feature, condition, W_f2h, W_c2h)

    agg12 = _prop_dual128(y12, src1, dst1)

    y3 = pl.pallas_call(
        _tc_b_body,
        out_shape=jax.ShapeDtypeStruct((N, HDIM), _f32),
    )(agg12, y12, dis, bf, bc, W_h2h)

    agg3 = _prop128(y3, src1, dst1)

    y4 = pl.pallas_call(
        _tc_c_body,
        out_shape=jax.ShapeDtypeStruct((N, HDIM), _f32),
    )(agg3[0], agg3[1], y3, dis, bh, W_h2l)

    agg4 = _prop128(y4, src1, dst1)

    z = pl.pallas_call(
        _tc_d_body,
        out_shape=jax.ShapeDtypeStruct((N, LDIM), _f32),
    )(agg4[0], agg4[1], y4, dis, bl)

    return z


# speedup vs baseline: 1.0022x; 1.0022x over previous
import functools

import jax
import jax.numpy as jnp
from jax import lax
from jax.experimental import pallas as pl
from jax.experimental.pallas import tpu as pltpu
from jax.experimental.pallas import tpu_sc as plsc

N = 10000
E = 320000
FDIM = 128
CDIM = 16
HDIM = 128
LDIM = 64

NC = 2
NS = 16
NW = NC * NS
E_PER = E // NW
K = 40
ITERS = E_PER // K
NB = 8
MAIN = (ITERS // NB) * NB
ROWB = 624
ROWB_LAST = N - (NS - 1) * ROWB
ZRB = 48

_f32 = jnp.float32


def _mesh():
    return plsc.VectorSubcoreMesh(core_axis_name="c", subcore_axis_name="s")


def _fill_zeros(zbuf, n_rows, C):
    def fz(i, _):
        def fz2(j, _):
            zbuf[i, pl.ds(j * 16, 16)] = jnp.zeros((16,), _f32)
            return 0

        lax.fori_loop(0, C // 16, fz2, 0)
        return 0

    lax.fori_loop(0, n_rows, fz, 0)


def _zero_acc(zbuf, acc, s):
    base_r = s * ROWB
    for i in range(ROWB // ZRB):
        pltpu.sync_copy(zbuf, acc.at[pl.ds(base_r + i * ZRB, ZRB)])

    @pl.when(s == NS - 1)
    def _():
        pltpu.sync_copy(zbuf.at[pl.ds(0, ROWB_LAST - ROWB)],
                        acc.at[pl.ds(base_r + ROWB, ROWB_LAST - ROWB)])


def _flush_acc(acc, out_hbm, c, s):
    base_r = s * ROWB

    @pl.when(s == NS - 1)
    def _():
        pltpu.sync_copy(acc.at[pl.ds(base_r, ROWB_LAST)],
                        out_hbm.at[c, pl.ds(base_r, ROWB_LAST)])

    @pl.when(s != NS - 1)
    def _():
        pltpu.sync_copy(acc.at[pl.ds(base_r, ROWB)],
                        out_hbm.at[c, pl.ds(base_r, ROWB)])


def _sc_degree(dst_hbm_arg):
    C = 128

    @functools.partial(
        pl.kernel,
        out_type=jax.ShapeDtypeStruct((NC, N, C), _f32),
        mesh=_mesh(),
        scratch_types=(
            [pltpu.VMEM((K,), jnp.int32) for _ in range(NB)]
            + [pltpu.VMEM((K, C), _f32), pltpu.VMEM((K, C), _f32)]
            + [pltpu.SemaphoreType.DMA for _ in range(2 * NB)]
            + [pltpu.VMEM_SHARED((N, C), _f32)]
        ),
    )
    def deg_kernel(dst_hbm, out_hbm, *scr):
        dv = scr[0:NB]
        ones_v = scr[NB]
        zbuf = scr[NB + 1]
        isem = scr[NB + 2:NB + 2 + NB]
        ssem = scr[NB + 2 + NB:NB + 2 + 2 * NB]
        acc = scr[-1]
        c = lax.axis_index("c")
        s = lax.axis_index("s")
        wid = c * NS + s

        def fill(i, _):
            def fill2(j, _):
                ones_v[i, pl.ds(j * 16, 16)] = jnp.ones((16,), _f32)
                return 0

            lax.fori_loop(0, C // 16, fill2, 0)
            return 0

        lax.fori_loop(0, K, fill, 0)
        _fill_zeros(zbuf, K, C)
        base_r = s * ROWB
        full = ROWB // K
        zd = []
        for i in range(full):
            zd.append(pltpu.async_copy(
                zbuf, acc.at[pl.ds(base_r + i * K, K)], isem[0]))
        for d in zd:
            d.wait()

        @pl.when(s == NS - 1)
        def _():
            for j in range((ROWB_LAST - full * K) // K):
                pltpu.sync_copy(zbuf,
                                acc.at[pl.ds(base_r + (full + j) * K, K)])

        @pl.when(s != NS - 1)
        def _():
            rem = ROWB - full * K
            if rem:
                pltpu.sync_copy(zbuf.at[pl.ds(0, rem)],
                                acc.at[pl.ds(base_r + full * K, rem)])

        plsc.subcore_barrier()

        base_e = wid * E_PER

        def body(g, _):
            ic = []
            for i in range(NB):
                ch = NB * g + i
                ic.append(pltpu.async_copy(
                    dst_hbm.at[pl.ds(base_e + ch * K, K)], dv[i], isem[i]))
            sc = []
            for i in range(NB):
                ic[i].wait()
                sc.append(pltpu.async_copy(ones_v, acc.at[dv[i]], ssem[i],
                                           add=True))
            for i in range(NB):
                sc[i].wait()
            return 0

        lax.fori_loop(0, ITERS // NB, body, 0)
        for ch in range(MAIN, ITERS):
            pltpu.sync_copy(dst_hbm.at[pl.ds(base_e + ch * K, K)], dv[0])
            pltpu.sync_copy(ones_v, acc.at[dv[0]], add=True)
        plsc.subcore_barrier()
        _flush_acc(acc, out_hbm, c, s)

    return deg_kernel(dst_hbm_arg)


def _make_prop(C):
    @functools.partial(
        pl.kernel,
        out_type=jax.ShapeDtypeStruct((NC, N, C), _f32),
        mesh=_mesh(),
        scratch_types=(
            [pltpu.VMEM((K,), jnp.int32) for _ in range(2 * NB)]
            + [pltpu.VMEM((K, C), _f32) for _ in range(NB)]
            + [pltpu.SemaphoreType.DMA for _ in range(3 * NB)]
            + [pltpu.VMEM_SHARED((N, C), _f32)]
        ),
    )
    def prop(y_hbm, src_hbm, dst_hbm, out_hbm, *scr):
        sv = scr[0:NB]
        dv = scr[NB:2 * NB]
        rows = scr[2 * NB:3 * NB]
        isem = scr[3 * NB:3 * NB + NB]
        gsem = scr[3 * NB + NB:3 * NB + 2 * NB]
        ssem = scr[3 * NB + 2 * NB:3 * NB + 3 * NB]
        acc = scr[-1]
        c = lax.axis_index("c")
        s = lax.axis_index("s")
        wid = c * NS + s

        _fill_zeros(rows[0], K, C)
        base_r = s * ROWB
        full = ROWB // K
        zd = []
        for i in range(full):
            zd.append(pltpu.async_copy(
                rows[0], acc.at[pl.ds(base_r + i * K, K)], gsem[0]))
        for d in zd:
            d.wait()

        @pl.when(s == NS - 1)
        def _():
            for j in range((ROWB_LAST - full * K) // K):
                pltpu.sync_copy(rows[0],
                                acc.at[pl.ds(base_r + (full + j) * K, K)])

        @pl.when(s != NS - 1)
        def _():
            rem = ROWB - full * K
            if rem:
                pltpu.sync_copy(rows[0].at[pl.ds(0, rem)],
                                acc.at[pl.ds(base_r + full * K, rem)])

        plsc.subcore_barrier()

        base_e = wid * E_PER

        def body(g, _):
            ic = []
            for i in range(NB):
                ch = NB * g + i
                off = base_e + ch * K
                i1 = pltpu.async_copy(src_hbm.at[pl.ds(off, K)], sv[i],
                                      isem[i])
                i2 = pltpu.async_copy(dst_hbm.at[pl.ds(off, K)], dv[i],
                                      isem[i])
                ic.append((i1, i2))
            gc = []
            for i in range(NB):
                ic[i][0].wait()
                ic[i][1].wait()
                gc.append(pltpu.async_copy(y_hbm.at[sv[i]], rows[i], gsem[i]))
            sc = []
            for i in range(NB):
                gc[i].wait()
                sc.append(pltpu.async_copy(rows[i], acc.at[dv[i]], ssem[i],
                                           add=True))
            for i in range(NB):
                sc[i].wait()
            return 0

        lax.fori_loop(0, ITERS // NB, body, 0)
        for ch in range(MAIN, ITERS):
            off = base_e + ch * K
            pltpu.sync_copy(src_hbm.at[pl.ds(off, K)], sv[0])
            pltpu.sync_copy(dst_hbm.at[pl.ds(off, K)], dv[0])
            pltpu.async_copy(y_hbm.at[sv[0]], rows[0], gsem[0]).wait()
            pltpu.sync_copy(rows[0], acc.at[dv[0]], add=True)
        plsc.subcore_barrier()
        _flush_acc(acc, out_hbm, c, s)

    return prop


def _make_prop_dual(C):
    E_PER2 = E // NS
    ITERS2 = E_PER2 // K
    MAIN2 = (ITERS2 // NB) * NB

    @functools.partial(
        pl.kernel,
        out_type=jax.ShapeDtypeStruct((NC, N, C), _f32),
        mesh=_mesh(),
        scratch_types=(
            [pltpu.VMEM((K,), jnp.int32) for _ in range(2 * NB)]
            + [pltpu.VMEM((K, C), _f32) for _ in range(NB)]
            + [pltpu.SemaphoreType.DMA for _ in range(3 * NB)]
            + [pltpu.VMEM_SHARED((N, C), _f32)]
        ),
    )
    def prop2(y_hbm, src_hbm, dst_hbm, out_hbm, *scr):
        sv = scr[0:NB]
        dv = scr[NB:2 * NB]
        rows = scr[2 * NB:3 * NB]
        isem = scr[3 * NB:3 * NB + NB]
        gsem = scr[3 * NB + NB:3 * NB + 2 * NB]
        ssem = scr[3 * NB + 2 * NB:3 * NB + 3 * NB]
        acc = scr[-1]
        c = lax.axis_index("c")
        s = lax.axis_index("s")

        _fill_zeros(rows[0], K, C)
        base_r = s * ROWB
        full = ROWB // K
        zd = []
        for i in range(full):
            zd.append(pltpu.async_copy(
                rows[0], acc.at[pl.ds(base_r + i * K, K)], gsem[0]))
        for d in zd:
            d.wait()

        @pl.when(s == NS - 1)
        def _():
            for j in range((ROWB_LAST - full * K) // K):
                pltpu.sync_copy(rows[0],
                                acc.at[pl.ds(base_r + (full + j) * K, K)])

        @pl.when(s != NS - 1)
        def _():
            rem = ROWB - full * K
            if rem:
                pltpu.sync_copy(rows[0].at[pl.ds(0, rem)],
                                acc.at[pl.ds(base_r + full * K, rem)])

        plsc.subcore_barrier()

        base_e = s * E_PER2
        ypanel = y_hbm.at[c]

        def body(g, _):
            ic = []
            for i in range(NB):
                ch = NB * g + i
                off = base_e + ch * K
                i1 = pltpu.async_copy(src_hbm.at[pl.ds(off, K)], sv[i],
                                      isem[i])
                i2 = pltpu.async_copy(dst_hbm.at[pl.ds(off, K)], dv[i],
                                      isem[i])
                ic.append((i1, i2))
            gc = []
            for i in range(NB):
                ic[i][0].wait()
                ic[i][1].wait()
                gc.append(pltpu.async_copy(ypanel.at[sv[i]], rows[i],
                                           gsem[i]))
            sc = []
            for i in range(NB):
                gc[i].wait()
                sc.append(pltpu.async_copy(rows[i], acc.at[dv[i]], ssem[i],
                                           add=True))
            for i in range(NB):
                sc[i].wait()
            return 0

        lax.fori_loop(0, ITERS2 // NB, body, 0)
        for ch in range(MAIN2, ITERS2):
            off = base_e + ch * K
            pltpu.sync_copy(src_hbm.at[pl.ds(off, K)], sv[0])
            pltpu.sync_copy(dst_hbm.at[pl.ds(off, K)], dv[0])
            pltpu.async_copy(ypanel.at[sv[0]], rows[0], gsem[0]).wait()
            pltpu.sync_copy(rows[0], acc.at[dv[0]], add=True)
        plsc.subcore_barrier()
        _flush_acc(acc, out_hbm, c, s)

    return prop2


_prop128 = _make_prop(HDIM)
_prop_dual128 = _make_prop_dual(HDIM)


def _tc_a0_body(f, cnd, w1, w2, xw_o):
    xw_o[0] = jnp.dot(f[...], w1[...], preferred_element_type=_f32)
    xw_o[1] = jnp.dot(cnd[...], w2[...], preferred_element_type=_f32)


def _tc_a1_body(degp, xw, dis_o, y12_o):
    dp = degp[...]
    deg = dp[0, :, 0:1] + dp[1, :, 0:1] + 1.0
    dis = lax.rsqrt(deg)
    dis_o[...] = dis
    y12_o[...] = dis[None] * xw[...]


def _tc_b_body(agg, y12, dis, bf, bc, w, y3_o):
    d = dis[...]
    a = agg[...]
    y = y12[...]
    ha = jnp.tanh(d * (a[0] + y[0]) + bf[...])
    hb = jnp.tanh(d * (a[1] + y[1]) + bc[...])
    wm = w[...]
    y3_o[...] = d * (jnp.dot(ha, wm[:HDIM], preferred_element_type=_f32)
                     + jnp.dot(hb, wm[HDIM:], preferred_element_type=_f32))


def _tc_c_body(a0, a1, y3, dis, bh, w, y4_o):
    d = dis[...]
    h3 = jnp.tanh(d * (a0[...] + a1[...] + y3[...]) + bh[...])
    y4 = d * jnp.dot(h3, w[...], preferred_element_type=_f32)
    y4_o[...] = jnp.concatenate([y4, jnp.zeros((N, HDIM - LDIM), _f32)], axis=1)


def _tc_d_body(a0, a1, y4, dis, bl, z_o):
    d = dis[...]
    z_o[...] = (d * (a0[...] + a1[...] + y4[...]))[:, :LDIM] + bl[...]


def kernel(feature, condition, edge_index, W_f2h, b_f2h, W_c2h, b_c2h,
           W_h2h, b_h2h, W_h2l, b_h2l):
    src1 = edge_index[0].astype(jnp.int32)
    dst1 = edge_index[1].astype(jnp.int32)
    bf = b_f2h.reshape(1, HDIM)
    bc = b_c2h.reshape(1, HDIM)
    bh = b_h2h.reshape(1, HDIM)
    bl = b_h2l.reshape(1, LDIM)

    degp = _sc_degree(dst1)

    xw12 = pl.pallas_call(
        _tc_a0_body,
        out_shape=jax.ShapeDtypeStruct((NC, N, HDIM), _f32),
    )(feature, condition, W_f2h, W_c2h)

    dis, y12 = pl.pallas_call(
        _tc_a1_body,
        out_shape=(
            jax.ShapeDtypeStruct((N, 1), _f32),
            jax.ShapeDtypeStruct((NC, N, HDIM), _f32),
        ),
    )(degp, xw12)

    agg12 = _prop_dual128(y12, src1, dst1)

    y3 = pl.pallas_call(
        _tc_b_body,
        out_shape=jax.ShapeDtypeStruct((N, HDIM), _f32),
    )(agg12, y12, dis, bf, bc, W_h2h)

    agg3 = _prop128(y3, src1, dst1)

    y4 = pl.pallas_call(
        _tc_c_body,
        out_shape=jax.ShapeDtypeStruct((N, HDIM), _f32),
    )(agg3[0], agg3[1], y3, dis, bh, W_h2l)

    agg4 = _prop128(y4, src1, dst1)

    z = pl.pallas_call(
        _tc_d_body,
        out_shape=jax.ShapeDtypeStruct((N, LDIM), _f32),
    )(agg4[0], agg4[1], y4, dis, bl)

    return z


# speedup vs baseline: 1.0024x; 1.0003x over previous
import functools

import jax
import jax.numpy as jnp
from jax import lax
from jax.experimental import pallas as pl
from jax.experimental.pallas import tpu as pltpu
from jax.experimental.pallas import tpu_sc as plsc

N = 10000
E = 320000
FDIM = 128
CDIM = 16
HDIM = 128
LDIM = 64

NC = 2
NS = 16
NW = NC * NS
E_PER = E // NW
K = 40
ITERS = E_PER // K
NB = 8
MAIN = (ITERS // NB) * NB
ROWB = 624
ROWB_LAST = N - (NS - 1) * ROWB

_f32 = jnp.float32


def _mesh():
    return plsc.VectorSubcoreMesh(core_axis_name="c", subcore_axis_name="s")


def _fill_zeros(zbuf, n_rows, C):
    def fz(i, _):
        def fz2(j, _):
            zbuf[i, pl.ds(j * 16, 16)] = jnp.zeros((16,), _f32)
            return 0

        lax.fori_loop(0, C // 16, fz2, 0)
        return 0

    lax.fori_loop(0, n_rows, fz, 0)


def _flush_acc(acc, out_hbm, c, s):
    base_r = s * ROWB

    @pl.when(s == NS - 1)
    def _():
        pltpu.sync_copy(acc.at[pl.ds(base_r, ROWB_LAST)],
                        out_hbm.at[c, pl.ds(base_r, ROWB_LAST)])

    @pl.when(s != NS - 1)
    def _():
        pltpu.sync_copy(acc.at[pl.ds(base_r, ROWB)],
                        out_hbm.at[c, pl.ds(base_r, ROWB)])


def _sc_degree(dst_hbm_arg):
    C = 128

    @functools.partial(
        pl.kernel,
        out_type=jax.ShapeDtypeStruct((NC, N, C), _f32),
        mesh=_mesh(),
        scratch_types=(
            [pltpu.VMEM((K,), jnp.int32) for _ in range(NB)]
            + [pltpu.VMEM((K, C), _f32), pltpu.VMEM((K, C), _f32)]
            + [pltpu.SemaphoreType.DMA for _ in range(2 * NB)]
            + [pltpu.VMEM_SHARED((N, C), _f32)]
        ),
    )
    def deg_kernel(dst_hbm, out_hbm, *scr):
        dv = scr[0:NB]
        ones_v = scr[NB]
        zbuf = scr[NB + 1]
        isem = scr[NB + 2:NB + 2 + NB]
        ssem = scr[NB + 2 + NB:NB + 2 + 2 * NB]
        acc = scr[-1]
        c = lax.axis_index("c")
        s = lax.axis_index("s")
        wid = c * NS + s

        def fill(i, _):
            def fill2(j, _):
                ones_v[i, pl.ds(j * 16, 16)] = jnp.ones((16,), _f32)
                return 0

            lax.fori_loop(0, C // 16, fill2, 0)
            return 0

        lax.fori_loop(0, K, fill, 0)
        _fill_zeros(zbuf, K, C)
        base_r = s * ROWB
        full = ROWB // K
        zd = []
        for i in range(full):
            zd.append(pltpu.async_copy(
                zbuf, acc.at[pl.ds(base_r + i * K, K)], isem[0]))
        for d in zd:
            d.wait()

        @pl.when(s == NS - 1)
        def _():
            for j in range((ROWB_LAST - full * K) // K):
                pltpu.sync_copy(zbuf,
                                acc.at[pl.ds(base_r + (full + j) * K, K)])

        @pl.when(s != NS - 1)
        def _():
            rem = ROWB - full * K
            if rem:
                pltpu.sync_copy(zbuf.at[pl.ds(0, rem)],
                                acc.at[pl.ds(base_r + full * K, rem)])

        plsc.subcore_barrier()

        base_e = wid * E_PER

        def body(g, _):
            ic = []
            for i in range(NB):
                ch = NB * g + i
                ic.append(pltpu.async_copy(
                    dst_hbm.at[pl.ds(base_e + ch * K, K)], dv[i], isem[i]))
            sc = []
            for i in range(NB):
                ic[i].wait()
                sc.append(pltpu.async_copy(ones_v, acc.at[dv[i]], ssem[i],
                                           add=True))
            for i in range(NB):
                sc[i].wait()
            return 0

        lax.fori_loop(0, ITERS // NB, body, 0)
        for ch in range(MAIN, ITERS):
            pltpu.sync_copy(dst_hbm.at[pl.ds(base_e + ch * K, K)], dv[0])
            pltpu.sync_copy(ones_v, acc.at[dv[0]], add=True)
        plsc.subcore_barrier()
        _flush_acc(acc, out_hbm, c, s)

    return deg_kernel(dst_hbm_arg)


def _make_prop(C):
    @functools.partial(
        pl.kernel,
        out_type=jax.ShapeDtypeStruct((NC, N, C), _f32),
        mesh=_mesh(),
        scratch_types=(
            [pltpu.VMEM((K,), jnp.int32) for _ in range(2 * NB)]
            + [pltpu.VMEM((K, C), _f32) for _ in range(NB)]
            + [pltpu.SemaphoreType.DMA for _ in range(3 * NB)]
            + [pltpu.VMEM_SHARED((N, C), _f32)]
        ),
    )
    def prop(y_hbm, src_hbm, dst_hbm, out_hbm, *scr):
        sv = scr[0:NB]
        dv = scr[NB:2 * NB]
        rows = scr[2 * NB:3 * NB]
        isem = scr[3 * NB:3 * NB + NB]
        gsem = scr[3 * NB + NB:3 * NB + 2 * NB]
        ssem = scr[3 * NB + 2 * NB:3 * NB + 3 * NB]
        acc = scr[-1]
        c = lax.axis_index("c")
        s = lax.axis_index("s")
        wid = c * NS + s

        _fill_zeros(rows[0], K, C)
        base_r = s * ROWB
        full = ROWB // K
        zd = []
        for i in range(full):
            zd.append(pltpu.async_copy(
                rows[0], acc.at[pl.ds(base_r + i * K, K)], gsem[0]))
        for d in zd:
            d.wait()

        @pl.when(s == NS - 1)
        def _():
            for j in range((ROWB_LAST - full * K) // K):
                pltpu.sync_copy(rows[0],
                                acc.at[pl.ds(base_r + (full + j) * K, K)])

        @pl.when(s != NS - 1)
        def _():
            rem = ROWB - full * K
            if rem:
                pltpu.sync_copy(rows[0].at[pl.ds(0, rem)],
                                acc.at[pl.ds(base_r + full * K, rem)])

        plsc.subcore_barrier()

        base_e = wid * E_PER

        def body(g, _):
            ic = []
            for i in range(NB):
                ch = NB * g + i
                off = base_e + ch * K
                i1 = pltpu.async_copy(src_hbm.at[pl.ds(off, K)], sv[i],
                                      isem[i])
                i2 = pltpu.async_copy(dst_hbm.at[pl.ds(off, K)], dv[i],
                                      isem[i])
                ic.append((i1, i2))
            gc = []
            for i in range(NB):
                ic[i][0].wait()
                ic[i][1].wait()
                gc.append(pltpu.async_copy(y_hbm.at[sv[i]], rows[i], gsem[i]))
            sc = []
            for i in range(NB):
                gc[i].wait()
                sc.append(pltpu.async_copy(rows[i], acc.at[dv[i]], ssem[i],
                                           add=True))
            for i in range(NB):
                sc[i].wait()
            return 0

        lax.fori_loop(0, ITERS // NB, body, 0)
        for ch in range(MAIN, ITERS):
            off = base_e + ch * K
            pltpu.sync_copy(src_hbm.at[pl.ds(off, K)], sv[0])
            pltpu.sync_copy(dst_hbm.at[pl.ds(off, K)], dv[0])
            pltpu.async_copy(y_hbm.at[sv[0]], rows[0], gsem[0]).wait()
            pltpu.sync_copy(rows[0], acc.at[dv[0]], add=True)
        plsc.subcore_barrier()
        _flush_acc(acc, out_hbm, c, s)

    return prop


def _make_prop_dual(C):
    E_PER2 = E // NS
    ITERS2 = E_PER2 // K
    MAIN2 = (ITERS2 // NB) * NB

    @functools.partial(
        pl.kernel,
        out_type=jax.ShapeDtypeStruct((NC, N, C), _f32),
        mesh=_mesh(),
        scratch_types=(
            [pltpu.VMEM((K,), jnp.int32) for _ in range(2 * NB)]
            + [pltpu.VMEM((K, C), _f32) for _ in range(NB)]
            + [pltpu.SemaphoreType.DMA for _ in range(3 * NB)]
            + [pltpu.VMEM_SHARED((N, C), _f32)]
        ),
    )
    def prop2(y_hbm, src_hbm, dst_hbm, out_hbm, *scr):
        sv = scr[0:NB]
        dv = scr[NB:2 * NB]
        rows = scr[2 * NB:3 * NB]
        isem = scr[3 * NB:3 * NB + NB]
        gsem = scr[3 * NB + NB:3 * NB + 2 * NB]
        ssem = scr[3 * NB + 2 * NB:3 * NB + 3 * NB]
        acc = scr[-1]
        c = lax.axis_index("c")
        s = lax.axis_index("s")

        _fill_zeros(rows[0], K, C)
        base_r = s * ROWB
        full = ROWB // K
        zd = []
        for i in range(full):
            zd.append(pltpu.async_copy(
                rows[0], acc.at[pl.ds(base_r + i * K, K)], gsem[0]))
        for d in zd:
            d.wait()

        @pl.when(s == NS - 1)
        def _():
            for j in range((ROWB_LAST - full * K) // K):
                pltpu.sync_copy(rows[0],
                                acc.at[pl.ds(base_r + (full + j) * K, K)])

        @pl.when(s != NS - 1)
        def _():
            rem = ROWB - full * K
            if rem:
                pltpu.sync_copy(rows[0].at[pl.ds(0, rem)],
                                acc.at[pl.ds(base_r + full * K, rem)])

        plsc.subcore_barrier()

        base_e = s * E_PER2
        ypanel = y_hbm.at[c]

        def body(g, _):
            ic = []
            for i in range(NB):
                ch = NB * g + i
                off = base_e + ch * K
                i1 = pltpu.async_copy(src_hbm.at[pl.ds(off, K)], sv[i],
                                      isem[i])
                i2 = pltpu.async_copy(dst_hbm.at[pl.ds(off, K)], dv[i],
                                      isem[i])
                ic.append((i1, i2))
            gc = []
            for i in range(NB):
                ic[i][0].wait()
                ic[i][1].wait()
                gc.append(pltpu.async_copy(ypanel.at[sv[i]], rows[i],
                                           gsem[i]))
            sc = []
            for i in range(NB):
                gc[i].wait()
                sc.append(pltpu.async_copy(rows[i], acc.at[dv[i]], ssem[i],
                                           add=True))
            for i in range(NB):
                sc[i].wait()
            return 0

        lax.fori_loop(0, ITERS2 // NB, body, 0)
        for ch in range(MAIN2, ITERS2):
            off = base_e + ch * K
            pltpu.sync_copy(src_hbm.at[pl.ds(off, K)], sv[0])
            pltpu.sync_copy(dst_hbm.at[pl.ds(off, K)], dv[0])
            pltpu.async_copy(ypanel.at[sv[0]], rows[0], gsem[0]).wait()
            pltpu.sync_copy(rows[0], acc.at[dv[0]], add=True)
        plsc.subcore_barrier()
        _flush_acc(acc, out_hbm, c, s)

    return prop2


_prop128 = _make_prop(HDIM)
_prop_dual128 = _make_prop_dual(HDIM)


def _tc_a0_body(f, cnd, w1, w2, xw_o):
    xw_o[0] = jnp.dot(f[...], w1[...], preferred_element_type=_f32)
    xw_o[1] = jnp.dot(cnd[...], w2[...], preferred_element_type=_f32)


def _tc_a1_body(degp, xw, dis_o, y12_o):
    dp = degp[...]
    deg = dp[0, :, 0:1] + dp[1, :, 0:1] + 1.0
    dis = lax.rsqrt(deg)
    dis_o[...] = dis
    y12_o[...] = dis[None] * xw[...]


def _tc_b_body(agg, y12, dis, bf, bc, w, y3_o):
    d = dis[...]
    a = agg[...]
    y = y12[...]
    ha = jnp.tanh(d * (a[0] + y[0]) + bf[...])
    hb = jnp.tanh(d * (a[1] + y[1]) + bc[...])
    wm = w[...]
    y3_o[...] = d * (jnp.dot(ha, wm[:HDIM], preferred_element_type=_f32)
                     + jnp.dot(hb, wm[HDIM:], preferred_element_type=_f32))


def _tc_c_body(a0, a1, y3, dis, bh, w, y4_o):
    d = dis[...]
    h3 = jnp.tanh(d * (a0[...] + a1[...] + y3[...]) + bh[...])
    y4 = d * jnp.dot(h3, w[...], preferred_element_type=_f32)
    y4_o[...] = jnp.concatenate([y4, jnp.zeros((N, HDIM - LDIM), _f32)], axis=1)


def _tc_d_body(a0, a1, y4, dis, bl, z_o):
    d = dis[...]
    z_o[...] = (d * (a0[...] + a1[...] + y4[...]))[:, :LDIM] + bl[...]


def kernel(feature, condition, edge_index, W_f2h, b_f2h, W_c2h, b_c2h,
           W_h2h, b_h2h, W_h2l, b_h2l):
    src1 = edge_index[0].astype(jnp.int32)
    dst1 = edge_index[1].astype(jnp.int32)
    bf = b_f2h.reshape(1, HDIM)
    bc = b_c2h.reshape(1, HDIM)
    bh = b_h2h.reshape(1, HDIM)
    bl = b_h2l.reshape(1, LDIM)

    degp = _sc_degree(dst1)

    xw12 = pl.pallas_call(
        _tc_a0_body,
        out_shape=jax.ShapeDtypeStruct((NC, N, HDIM), _f32),
    )(feature, condition, W_f2h, W_c2h)

    dis, y12 = pl.pallas_call(
        _tc_a1_body,
        out_shape=(
            jax.ShapeDtypeStruct((N, 1), _f32),
            jax.ShapeDtypeStruct((NC, N, HDIM), _f32),
        ),
    )(degp, xw12)

    agg12 = _prop_dual128(y12, src1, dst1)

    y3 = pl.pallas_call(
        _tc_b_body,
        out_shape=jax.ShapeDtypeStruct((N, HDIM), _f32),
    )(agg12, y12, dis, bf, bc, W_h2h)

    agg3 = _prop128(y3, src1, dst1)

    y4 = pl.pallas_call(
        _tc_c_body,
        out_shape=jax.ShapeDtypeStruct((N, HDIM), _f32),
    )(agg3[0], agg3[1], y3, dis, bh, W_h2l)

    agg4 = _prop128(y4, src1, dst1)

    z = pl.pallas_call(
        _tc_d_body,
        out_shape=jax.ShapeDtypeStruct((N, LDIM), _f32),
    )(agg4[0], agg4[1], y4, dis, bl)

    return z
